# baseline (speedup 1.0000x reference)
import functools

import jax
import jax.numpy as jnp
from jax import lax
from jax.experimental import pallas as pl
from jax.experimental.pallas import tpu as pltpu
from jax.experimental.pallas import tpu_sc as plsc

B = 16384
D = 64
D2 = 2 * D
R = 1000000
NG = 4 * B
NC, NS = 2, 16
NW = NC * NS
ROWS_PER_W = NG // NW
CHUNK = 512
N_CHUNK = ROWS_PER_W // CHUNK

_sc_mesh = plsc.VectorSubcoreMesh(core_axis_name="c", subcore_axis_name="s")


@functools.partial(
    pl.kernel,
    mesh=_sc_mesh,
    out_type=jax.ShapeDtypeStruct((NG, D2), jnp.float32),
    scratch_types=[
        pltpu.VMEM((CHUNK,), jnp.int32),
        pltpu.VMEM((CHUNK, D2), jnp.float32),
        pltpu.SemaphoreType.DMA,
    ],
)
def _sc_gather(table_hbm, idx_hbm, out_hbm, idx_v, rows_v, sem):
    wid = lax.axis_index("s") * NC + lax.axis_index("c")
    base = wid * ROWS_PER_W

    @pl.loop(0, N_CHUNK)
    def _(c):
        off = base + c * CHUNK
        pltpu.sync_copy(idx_hbm.at[pl.ds(off, CHUNK)], idx_v)
        pltpu.async_copy(table_hbm.at[idx_v], rows_v, sem).wait()
        pltpu.sync_copy(rows_v, out_hbm.at[pl.ds(off, CHUNK)])


BB = 2048
NB = B // BB


def _loss_body(pos_ref, neg_ref, p0_ref, p1_ref, pr_ref, out_ref):
    pos = pos_ref[:, :D]
    neg = neg_ref[:, :D]
    ps = p0_ref[:, :D] + p1_ref[:, :D]
    pos_n = jnp.sum(jnp.abs(pos - ps), axis=1)
    neg_n = jnp.sum(jnp.abs(neg - ps), axis=1)
    pr = pr_ref[...][:, 0]
    diff = 1.0 + pr * pos_n - neg_n
    part = jnp.sum(jnp.maximum(diff, 0.0))

    @pl.when(pl.program_id(0) == 0)
    def _():
        out_ref[0, 0] = 0.0

    out_ref[0, 0] += part


_loss_call = pl.pallas_call(
    _loss_body,
    grid=(NB,),
    in_specs=[
        pl.BlockSpec((BB, D2), lambda i: (i, 0)),
        pl.BlockSpec((BB, D2), lambda i: (i + NB, 0)),
        pl.BlockSpec((BB, D2), lambda i: (i + 2 * NB, 0)),
        pl.BlockSpec((BB, D2), lambda i: (i + 3 * NB, 0)),
        pl.BlockSpec((BB, 1), lambda i: (i, 0)),
    ],
    out_specs=pl.BlockSpec((1, 1), lambda i: (0, 0),
                           memory_space=pltpu.SMEM),
    out_shape=jax.ShapeDtypeStruct((1, 1), jnp.float32),
)


def kernel(rel, rel_neg, path_rel, pr, relation_emb):
    idx = jnp.concatenate([
        rel.astype(jnp.int32),
        rel_neg.astype(jnp.int32),
        path_rel[:, 0].astype(jnp.int32),
        path_rel[:, 1].astype(jnp.int32),
    ])
    table2 = jnp.pad(relation_emb, ((0, 0), (0, D)))
    gathered = _sc_gather(table2, idx)
    out = _loss_call(gathered, gathered, gathered, gathered,
                     pr.reshape(B, 1))
    return out[0, 0]


# speedup vs baseline: 1.4916x; 1.4916x over previous
import functools

import jax
import jax.numpy as jnp
from jax import lax
from jax.experimental import pallas as pl
from jax.experimental.pallas import tpu as pltpu
from jax.experimental.pallas import tpu_sc as plsc

B = 16384
D = 64
D2 = 2 * D
R = 1000000
NG = 4 * B
NC, NS = 2, 16
NW = NC * NS
ROWS_PER_W = NG // NW
CHUNK = 256
N_CHUNK = ROWS_PER_W // CHUNK

_sc_mesh = plsc.VectorSubcoreMesh(core_axis_name="c", subcore_axis_name="s")


@functools.partial(
    pl.kernel,
    mesh=_sc_mesh,
    out_type=jax.ShapeDtypeStruct((NG, D2), jnp.float32),
    scratch_types=[
        pltpu.VMEM((ROWS_PER_W,), jnp.int32),
        pltpu.VMEM((CHUNK, D2), jnp.float32),
        pltpu.VMEM((CHUNK, D2), jnp.float32),
        pltpu.SemaphoreType.DMA,
        pltpu.SemaphoreType.DMA,
    ],
)
def _sc_gather(table_hbm, idx_hbm, out_hbm, idx_v, rows_v0, rows_v1,
               sem0, sem1):
    wid = lax.axis_index("s") * NC + lax.axis_index("c")
    base = wid * ROWS_PER_W
    pltpu.sync_copy(idx_hbm.at[pl.ds(base, ROWS_PER_W)], idx_v)

    bufs = (rows_v0, rows_v1)
    sems = (sem0, sem1)

    def start(c):
        pltpu.async_copy(
            table_hbm.at[idx_v.at[pl.ds(c * CHUNK, CHUNK)]],
            bufs[c % 2], sems[c % 2])

    start(0)
    for c in range(N_CHUNK):
        if c + 1 < N_CHUNK:
            start(c + 1)
        pltpu.make_async_copy(
            table_hbm.at[idx_v.at[pl.ds(c * CHUNK, CHUNK)]],
            bufs[c % 2], sems[c % 2]).wait()
        pltpu.sync_copy(bufs[c % 2],
                        out_hbm.at[pl.ds(base + c * CHUNK, CHUNK)])


TW = 4096
NT = (R + TW - 1) // TW


def _tr_body(xt_ref, out_ref):
    x = xt_ref[...]
    out_ref[:, :D] = jnp.transpose(x)


_tr_call = pl.pallas_call(
    _tr_body,
    grid=(NT,),
    in_specs=[pl.BlockSpec((D, TW), lambda i: (0, i))],
    out_specs=pl.BlockSpec((TW, D2), lambda i: (i, 0)),
    out_shape=jax.ShapeDtypeStruct((NT * TW, D2), jnp.float32),
)


BB = 2048
NB = B // BB


def _loss_body(pos_ref, neg_ref, p0_ref, p1_ref, pr_ref, out_ref):
    pos = pos_ref[:, :D]
    neg = neg_ref[:, :D]
    ps = p0_ref[:, :D] + p1_ref[:, :D]
    pos_n = jnp.sum(jnp.abs(pos - ps), axis=1)
    neg_n = jnp.sum(jnp.abs(neg - ps), axis=1)
    pr = pr_ref[...][:, 0]
    diff = 1.0 + pr * pos_n - neg_n
    part = jnp.sum(jnp.maximum(diff, 0.0))

    @pl.when(pl.program_id(0) == 0)
    def _():
        out_ref[0, 0] = 0.0

    out_ref[0, 0] += part


_loss_call = pl.pallas_call(
    _loss_body,
    grid=(NB,),
    in_specs=[
        pl.BlockSpec((BB, D2), lambda i: (i, 0)),
        pl.BlockSpec((BB, D2), lambda i: (i + NB, 0)),
        pl.BlockSpec((BB, D2), lambda i: (i + 2 * NB, 0)),
        pl.BlockSpec((BB, D2), lambda i: (i + 3 * NB, 0)),
        pl.BlockSpec((BB, 1), lambda i: (i, 0)),
    ],
    out_specs=pl.BlockSpec((1, 1), lambda i: (0, 0),
                           memory_space=pltpu.SMEM),
    out_shape=jax.ShapeDtypeStruct((1, 1), jnp.float32),
)


def kernel(rel, rel_neg, path_rel, pr, relation_emb):
    idx = jnp.concatenate([
        rel.astype(jnp.int32),
        rel_neg.astype(jnp.int32),
        path_rel[:, 0].astype(jnp.int32),
        path_rel[:, 1].astype(jnp.int32),
    ])
    table2 = _tr_call(relation_emb.T)
    gathered = _sc_gather(table2, idx)
    out = _loss_call(gathered, gathered, gathered, gathered,
                     pr.reshape(B, 1))
    return out[0, 0]


# speedup vs baseline: 1.8169x; 1.2181x over previous
import functools

import jax
import jax.numpy as jnp
from jax import lax
from jax.experimental import pallas as pl
from jax.experimental.pallas import tpu as pltpu
from jax.experimental.pallas import tpu_sc as plsc

B = 16384
D = 64
D2 = 2 * D
R = 1000000
NG = 4 * B
NC, NS = 2, 16
NW = NC * NS
ROWS_PER_W = NG // NW
CHUNK = 256
N_CHUNK = ROWS_PER_W // CHUNK

_sc_mesh = plsc.VectorSubcoreMesh(core_axis_name="c", subcore_axis_name="s")


@functools.partial(
    pl.kernel,
    mesh=_sc_mesh,
    out_type=jax.ShapeDtypeStruct((NG, D2), jnp.float32),
    scratch_types=[
        pltpu.VMEM((ROWS_PER_W,), jnp.int32),
        pltpu.VMEM((CHUNK, D2), jnp.float32),
        pltpu.VMEM((CHUNK, D2), jnp.float32),
        pltpu.SemaphoreType.DMA,
        pltpu.SemaphoreType.DMA,
    ],
)
def _sc_gather(table_hbm, idx_hbm, out_hbm, idx_v, rows_v0, rows_v1,
               sem0, sem1):
    wid = lax.axis_index("s") * NC + lax.axis_index("c")
    base = wid * ROWS_PER_W
    pltpu.sync_copy(idx_hbm.at[pl.ds(base, ROWS_PER_W)], idx_v)

    bufs = (rows_v0, rows_v1)
    sems = (sem0, sem1)

    def start(c):
        pltpu.async_copy(
            table_hbm.at[idx_v.at[pl.ds(c * CHUNK, CHUNK)]],
            bufs[c % 2], sems[c % 2])

    start(0)
    for c in range(N_CHUNK):
        if c + 1 < N_CHUNK:
            start(c + 1)
        pltpu.make_async_copy(
            table_hbm.at[idx_v.at[pl.ds(c * CHUNK, CHUNK)]],
            bufs[c % 2], sems[c % 2]).wait()
        pltpu.sync_copy(bufs[c % 2],
                        out_hbm.at[pl.ds(base + c * CHUNK, CHUNK)])


TW = 8192
NT = (R + TW - 1) // TW


def _tr_body(xt_ref, out_ref):
    x = xt_ref[...]
    out_ref[:, :D] = jnp.transpose(x)


_tr_call = pl.pallas_call(
    _tr_body,
    grid=(NT,),
    in_specs=[pl.BlockSpec((D, TW), lambda i: (0, i))],
    out_specs=pl.BlockSpec((TW, D2), lambda i: (i, 0)),
    out_shape=jax.ShapeDtypeStruct((NT * TW, D2), jnp.float32),
)


BB = 2048
NB = B // BB


def _loss_body(pos_ref, neg_ref, p0_ref, p1_ref, pr_ref, out_ref):
    pos = pos_ref[:, :D]
    neg = neg_ref[:, :D]
    ps = p0_ref[:, :D] + p1_ref[:, :D]
    pos_n = jnp.sum(jnp.abs(pos - ps), axis=1)
    neg_n = jnp.sum(jnp.abs(neg - ps), axis=1)
    pr = pr_ref[...][:, 0]
    diff = 1.0 + pr * pos_n - neg_n
    part = jnp.sum(jnp.maximum(diff, 0.0))

    @pl.when(pl.program_id(0) == 0)
    def _():
        out_ref[0, 0] = 0.0

    out_ref[0, 0] += part


_loss_call = pl.pallas_call(
    _loss_body,
    grid=(NB,),
    in_specs=[
        pl.BlockSpec((BB, D2), lambda i: (i, 0)),
        pl.BlockSpec((BB, D2), lambda i: (i + NB, 0)),
        pl.BlockSpec((BB, D2), lambda i: (i + 2 * NB, 0)),
        pl.BlockSpec((BB, D2), lambda i: (i + 3 * NB, 0)),
        pl.BlockSpec((BB, 1), lambda i: (i, 0)),
    ],
    out_specs=pl.BlockSpec((1, 1), lambda i: (0, 0),
                           memory_space=pltpu.SMEM),
    out_shape=jax.ShapeDtypeStruct((1, 1), jnp.float32),
)


def kernel(rel, rel_neg, path_rel, pr, relation_emb):
    idx = jnp.concatenate([
        rel.astype(jnp.int32),
        rel_neg.astype(jnp.int32),
        path_rel[:, 0].astype(jnp.int32),
        path_rel[:, 1].astype(jnp.int32),
    ])
    table2 = _tr_call(relation_emb.T)
    gathered = _sc_gather(table2, idx)
    out = _loss_call(gathered, gathered, gathered, gathered,
                     pr.reshape(B, 1))
    return out[0, 0]
